# baseline (speedup 1.0000x reference)
import functools

import jax
import jax.numpy as jnp
from jax import lax
from jax.experimental import pallas as pl
from jax.experimental.pallas import tpu as pltpu
from jax.experimental.pallas import tpu_sc as plsc

D = 32
K = 1024
BT = 2048
N_TOK = 64 * 1024
GRID = N_TOK // BT
ZROWS = BT // 1024
LOSS_SCALE = 1.25 / (N_TOK * D)


_ROW_BIAS = 0x3F800000


def _argmin_body(z_ref, cb_ref, c2_ref, idx_ref, loss_ref, acc_ref, rows_ref):
    @pl.when(pl.program_id(0) == 0)
    def _():
        r = lax.broadcasted_iota(jnp.int32, (K, BT), 0) + _ROW_BIAS
        rows_ref[...] = lax.bitcast_convert_type(r, jnp.float32)
        acc_ref[...] = jnp.zeros_like(acc_ref)

    z = z_ref[...].reshape(BT, D)
    cb = cb_ref[...]
    z2 = jnp.sum(z * z, axis=1, keepdims=True)
    zc_t = jnp.dot(cb, z.T, preferred_element_type=jnp.float32)
    dist_t = z2.T - 2.0 * zc_t + c2_ref[...]
    minval = jnp.min(dist_t, axis=0, keepdims=True)
    rf = jnp.min(jnp.where(dist_t == minval, rows_ref[...], 2.0), axis=0)
    idx_ref[...] = lax.bitcast_convert_type(rf, jnp.int32) - _ROW_BIAS

    acc_ref[...] += minval

    @pl.when(pl.program_id(0) == GRID - 1)
    def _():
        loss_ref[...] = jnp.sum(acc_ref[...]).reshape(1, 1) * LOSS_SCALE


def _tc_argmin(z3, codebook, c2):
    return pl.pallas_call(
        _argmin_body,
        grid=(GRID,),
        in_specs=[
            pl.BlockSpec((ZROWS, 1024, D), lambda i: (i, 0, 0)),
            pl.BlockSpec((K, D), lambda i: (0, 0)),
            pl.BlockSpec((K, 1), lambda i: (0, 0)),
        ],
        out_specs=[
            pl.BlockSpec((BT,), lambda i: (i,)),
            pl.BlockSpec((1, 1), lambda i: (0, 0)),
        ],
        out_shape=[
            jax.ShapeDtypeStruct((N_TOK,), jnp.int32),
            jax.ShapeDtypeStruct((1, 1), jnp.float32),
        ],
        scratch_shapes=[
            pltpu.VMEM((1, BT), jnp.float32),
            pltpu.VMEM((K, BT), jnp.float32),
        ],
    )(z3, codebook, c2)


_NC, _NS = 2, 16
NW = _NC * _NS
B_PER_W = N_TOK // NW
CHUNK = 128
N_CHUNK = B_PER_W // CHUNK

@functools.cache
def _make_sc_gather():
    mesh = plsc.VectorSubcoreMesh(
        core_axis_name="c", subcore_axis_name="s", num_cores=_NC, num_subcores=_NS
    )

    @functools.partial(
        pl.kernel,
        out_type=jax.ShapeDtypeStruct((N_TOK, D), jnp.float32),
        mesh=mesh,
        scratch_types=[
            pltpu.VMEM((B_PER_W,), jnp.int32),
            pltpu.VMEM((B_PER_W, D), jnp.float32),
            pltpu.SemaphoreType.DMA,
        ],
        compiler_params=pltpu.CompilerParams(use_tc_tiling_on_sc=False),
    )
    def _sc_gather(cb_hbm, idx_hbm, out_hbm, idx_v, rows_v, sem):
        wid = lax.axis_index("s") * _NC + lax.axis_index("c")
        base = wid * B_PER_W
        pltpu.sync_copy(idx_hbm.at[pl.ds(base, B_PER_W)], idx_v)
        copies = [
            pltpu.async_copy(
                cb_hbm.at[idx_v.at[pl.ds(j * CHUNK, CHUNK)]],
                rows_v.at[pl.ds(j * CHUNK, CHUNK)],
                sem,
            )
            for j in range(N_CHUNK)
        ]
        for c in copies:
            c.wait()
        pltpu.sync_copy(rows_v, out_hbm.at[pl.ds(base, B_PER_W)])

    return _sc_gather


def kernel(z, codebook):
    c2 = jnp.sum(codebook**2, axis=1, keepdims=True)
    flat_idx, loss = _tc_argmin(z, codebook, c2)
    z_q_flat = _make_sc_gather()(codebook, flat_idx)
    z_q = z_q_flat.reshape(z.shape)
    indices = flat_idx.reshape(z.shape[:-1])
    vq_loss = loss.reshape(())
    return (z_q, indices, vq_loss)


# speedup vs baseline: 1.0034x; 1.0034x over previous
import functools

import jax
import jax.numpy as jnp
from jax import lax
from jax.experimental import pallas as pl
from jax.experimental.pallas import tpu as pltpu
from jax.experimental.pallas import tpu_sc as plsc

D = 32
K = 1024
BT = 1024
N_TOK = 64 * 1024
GRID = N_TOK // BT
ZROWS = BT // 1024
LOSS_SCALE = 1.25 / (N_TOK * D)


_ROW_BIAS = 0x3F800000


def _argmin_body(z_ref, cb_ref, c2_ref, idx_ref, loss_ref, acc_ref, rows_ref):
    @pl.when(pl.program_id(0) == 0)
    def _():
        r = lax.broadcasted_iota(jnp.int32, (K, BT), 0) + _ROW_BIAS
        rows_ref[...] = lax.bitcast_convert_type(r, jnp.float32)
        acc_ref[...] = jnp.zeros_like(acc_ref)

    z = z_ref[...].reshape(BT, D)
    cb = cb_ref[...]
    z2 = jnp.sum(z * z, axis=1, keepdims=True)
    zc_t = jnp.dot(cb, z.T, preferred_element_type=jnp.float32)
    dist_t = z2.T - 2.0 * zc_t + c2_ref[...]
    minval = jnp.min(dist_t, axis=0, keepdims=True)
    rf = jnp.min(jnp.where(dist_t == minval, rows_ref[...], 2.0), axis=0)
    idx_ref[...] = lax.bitcast_convert_type(rf, jnp.int32) - _ROW_BIAS

    acc_ref[...] += minval

    @pl.when(pl.program_id(0) == GRID - 1)
    def _():
        loss_ref[...] = jnp.sum(acc_ref[...]).reshape(1, 1) * LOSS_SCALE


def _tc_argmin(z3, codebook, c2):
    return pl.pallas_call(
        _argmin_body,
        grid=(GRID,),
        in_specs=[
            pl.BlockSpec((ZROWS, 1024, D), lambda i: (i, 0, 0)),
            pl.BlockSpec((K, D), lambda i: (0, 0)),
            pl.BlockSpec((K, 1), lambda i: (0, 0)),
        ],
        out_specs=[
            pl.BlockSpec((BT,), lambda i: (i,)),
            pl.BlockSpec((1, 1), lambda i: (0, 0)),
        ],
        out_shape=[
            jax.ShapeDtypeStruct((N_TOK,), jnp.int32),
            jax.ShapeDtypeStruct((1, 1), jnp.float32),
        ],
        scratch_shapes=[
            pltpu.VMEM((1, BT), jnp.float32),
            pltpu.VMEM((K, BT), jnp.float32),
        ],
    )(z3, codebook, c2)


_NC, _NS = 2, 16
NW = _NC * _NS
B_PER_W = N_TOK // NW
CHUNK = 128
N_CHUNK = B_PER_W // CHUNK

@functools.cache
def _make_sc_gather():
    mesh = plsc.VectorSubcoreMesh(
        core_axis_name="c", subcore_axis_name="s", num_cores=_NC, num_subcores=_NS
    )

    @functools.partial(
        pl.kernel,
        out_type=jax.ShapeDtypeStruct((N_TOK, D), jnp.float32),
        mesh=mesh,
        scratch_types=[
            pltpu.VMEM((B_PER_W,), jnp.int32),
            pltpu.VMEM((B_PER_W, D), jnp.float32),
            pltpu.SemaphoreType.DMA,
        ],
        compiler_params=pltpu.CompilerParams(use_tc_tiling_on_sc=False),
    )
    def _sc_gather(cb_hbm, idx_hbm, out_hbm, idx_v, rows_v, sem):
        wid = lax.axis_index("s") * _NC + lax.axis_index("c")
        base = wid * B_PER_W
        pltpu.sync_copy(idx_hbm.at[pl.ds(base, B_PER_W)], idx_v)
        copies = [
            pltpu.async_copy(
                cb_hbm.at[idx_v.at[pl.ds(j * CHUNK, CHUNK)]],
                rows_v.at[pl.ds(j * CHUNK, CHUNK)],
                sem,
            )
            for j in range(N_CHUNK)
        ]
        for c in copies:
            c.wait()
        pltpu.sync_copy(rows_v, out_hbm.at[pl.ds(base, B_PER_W)])

    return _sc_gather


def kernel(z, codebook):
    c2 = jnp.sum(codebook**2, axis=1, keepdims=True)
    flat_idx, loss = _tc_argmin(z, codebook, c2)
    z_q_flat = _make_sc_gather()(codebook, flat_idx)
    z_q = z_q_flat.reshape(z.shape)
    indices = flat_idx.reshape(z.shape[:-1])
    vq_loss = loss.reshape(())
    return (z_q, indices, vq_loss)


# speedup vs baseline: 1.0939x; 1.0902x over previous
import functools

import jax
import jax.numpy as jnp
from jax import lax
from jax.experimental import pallas as pl
from jax.experimental.pallas import tpu as pltpu
from jax.experimental.pallas import tpu_sc as plsc

D = 32
K = 1024
BT = 1024
N_TOK = 64 * 1024
GRID = N_TOK // BT
ZROWS = BT // 1024
LOSS_SCALE = 1.25 / (N_TOK * D)


_ROW_BIAS = 0x3F800000


def _argmin_body(z_ref, cbm2_ref, c2_ref, idx_ref, loss_ref, acc_ref, rows_ref):
    @pl.when(pl.program_id(0) == 0)
    def _():
        r = lax.broadcasted_iota(jnp.int32, (K, BT), 0) + _ROW_BIAS
        rows_ref[...] = lax.bitcast_convert_type(r, jnp.float32)
        acc_ref[...] = jnp.zeros_like(acc_ref)

    z = z_ref[...]
    cbm2 = cbm2_ref[...]
    z2 = jnp.sum(z * z, axis=1, keepdims=True)
    m2zc_t = jnp.dot(cbm2, z.T, preferred_element_type=jnp.float32)
    dist_t = z2.T + m2zc_t + c2_ref[...]
    minval = jnp.min(dist_t, axis=0, keepdims=True)
    rf = jnp.min(jnp.where(dist_t == minval, rows_ref[...], 2.0), axis=0)
    idx_ref[...] = lax.bitcast_convert_type(rf, jnp.int32) - _ROW_BIAS

    acc_ref[...] += minval

    @pl.when(pl.program_id(0) == GRID - 1)
    def _():
        loss_ref[...] = jnp.sum(acc_ref[...]).reshape(1, 1) * LOSS_SCALE


def _tc_argmin(flat_z, cbm2, c2):
    return pl.pallas_call(
        _argmin_body,
        grid=(GRID,),
        in_specs=[
            pl.BlockSpec((BT, D), lambda i: (i, 0)),
            pl.BlockSpec((K, D), lambda i: (0, 0)),
            pl.BlockSpec((K, 1), lambda i: (0, 0)),
        ],
        out_specs=[
            pl.BlockSpec((BT,), lambda i: (i,)),
            pl.BlockSpec((1, 1), lambda i: (0, 0)),
        ],
        out_shape=[
            jax.ShapeDtypeStruct((N_TOK,), jnp.int32),
            jax.ShapeDtypeStruct((1, 1), jnp.float32),
        ],
        scratch_shapes=[
            pltpu.VMEM((1, BT), jnp.float32),
            pltpu.VMEM((K, BT), jnp.float32),
        ],
    )(flat_z, cbm2, c2)


_NC, _NS = 2, 16
NW = _NC * _NS
B_PER_W = N_TOK // NW
CHUNK = 128
N_CHUNK = B_PER_W // CHUNK

@functools.cache
def _make_sc_gather():
    mesh = plsc.VectorSubcoreMesh(
        core_axis_name="c", subcore_axis_name="s", num_cores=_NC, num_subcores=_NS
    )

    @functools.partial(
        pl.kernel,
        out_type=jax.ShapeDtypeStruct((N_TOK, D), jnp.float32),
        mesh=mesh,
        scratch_types=[
            pltpu.VMEM((B_PER_W,), jnp.int32),
            pltpu.VMEM((B_PER_W, D), jnp.float32),
            pltpu.SemaphoreType.DMA,
            pltpu.SemaphoreType.DMA,
        ],
        compiler_params=pltpu.CompilerParams(use_tc_tiling_on_sc=False),
    )
    def _sc_gather(cb_hbm, idx_hbm, out_hbm, idx_v, rows_v, sem, osem):
        wid = lax.axis_index("s") * _NC + lax.axis_index("c")
        base = wid * B_PER_W
        pltpu.sync_copy(idx_hbm.at[pl.ds(base, B_PER_W)], idx_v)
        copies = [
            pltpu.async_copy(
                cb_hbm.at[idx_v.at[pl.ds(j * CHUNK, CHUNK)]],
                rows_v.at[pl.ds(j * CHUNK, CHUNK)],
                sem,
            )
            for j in range(N_CHUNK)
        ]
        outs = []
        for j, c in enumerate(copies):
            c.wait()
            outs.append(
                pltpu.async_copy(
                    rows_v.at[pl.ds(j * CHUNK, CHUNK)],
                    out_hbm.at[pl.ds(base + j * CHUNK, CHUNK)],
                    osem,
                )
            )
        for o in outs:
            o.wait()

    return _sc_gather


def kernel(z, codebook):
    flat_z = z.reshape(N_TOK, D)
    c2 = jnp.sum(codebook**2, axis=1, keepdims=True)
    flat_idx, loss = _tc_argmin(flat_z, -2.0 * codebook, c2)
    z_q_flat = _make_sc_gather()(codebook, flat_idx)
    z_q = z_q_flat.reshape(z.shape)
    indices = flat_idx.reshape(z.shape[:-1])
    vq_loss = loss.reshape(())
    return (z_q, indices, vq_loss)
